# Initial kernel scaffold; baseline (speedup 1.0000x reference)
import jax
import jax.numpy as jnp
from jax.experimental import pallas as pl


def kernel(x, edge_index, batch, params):
    raise NotImplementedError("write your pallas kernel here")


# speedup vs baseline: 30.1389x; 30.1389x over previous
import functools

import jax
import jax.numpy as jnp
from jax import lax
from jax.experimental import pallas as pl
from jax.experimental.pallas import tpu as pltpu
from jax.experimental.pallas import tpu_sc as plsc

N = 10000
NP = 10240
E = 160000
H = 32
F = 128
NC = 2
NS = 16
NW = NC * NS
CH = 128
CPW = 40
EP = NW * CPW * CH
RPS = NP // NS
PK = 4
PR = NP // PK
BLK = 128
GRID = PR // BLK
INV_H = 13.0 / 6.0
PREC = lax.Precision.HIGHEST


def _spline_tab(coeff, blend):
    cjb = coeff.T.astype(jnp.bfloat16).astype(jnp.float32)
    return (jnp.tile(cjb, (1, PK)),
            jnp.tile(jnp.abs(blend)[None, :].astype(jnp.float32), (1, PK)))


def _spline(x, cjb, blend):
    s = (jnp.clip(x, -3.0, 3.0) + 3.0) * INV_H
    acc = jnp.zeros_like(x)
    for j in range(11):
        u = jnp.clip(s - float(j), 0.0, 3.0)
        q = jnp.where(u < 1.0, 0.5 * u * u,
                      jnp.where(u < 2.0, (3.0 - u) * u - 1.5,
                                0.5 * (3.0 - u) * (3.0 - u)))
        qb = q.astype(jnp.bfloat16).astype(jnp.float32)
        acc = acc + qb * cjb[j:j + 1, :]
    return x * jax.nn.sigmoid(x) + blend * acc


def _dot(x, w):
    return jnp.dot(x.astype(jnp.bfloat16), w.astype(jnp.bfloat16),
                   preferred_element_type=jnp.float32)


def _dot_exact(x, w):
    return jnp.dot(x, w, preferred_element_type=jnp.float32, precision=PREC)


def _mesh():
    return plsc.VectorSubcoreMesh(core_axis_name="c", subcore_axis_name="s",
                                  num_cores=NC, num_subcores=NS)


_SC_PARAMS = pltpu.CompilerParams(use_tc_tiling_on_sc=False)


def _sc_count(col3, z32, ones32):
    @functools.partial(
        pl.kernel,
        out_type=jax.ShapeDtypeStruct((NC, NP, H), jnp.float32),
        mesh=_mesh(),
        compiler_params=_SC_PARAMS,
        scratch_types=[
            pltpu.VMEM((CPW, CH), jnp.int32),
            pltpu.VMEM((CH, H), jnp.float32),
            pltpu.VMEM_SHARED((NP, H), jnp.float32),
        ],
    )
    def k(col_hbm, z_hbm, ones_hbm, out_hbm, colv, onesv, shared):
        cid = lax.axis_index("c")
        sid = lax.axis_index("s")
        wid = sid * NC + cid
        base = sid * RPS
        pltpu.sync_copy(z_hbm.at[pl.ds(base, RPS)], shared.at[pl.ds(base, RPS)])
        pltpu.sync_copy(ones_hbm, onesv)
        pltpu.sync_copy(col_hbm.at[wid], colv)
        plsc.subcore_barrier()

        @pl.loop(0, CPW)
        def _(j):
            pltpu.sync_copy(onesv, shared.at[colv.at[j]], add=True)

        plsc.subcore_barrier()
        pltpu.sync_copy(shared.at[pl.ds(base, RPS)],
                        out_hbm.at[cid, pl.ds(base, RPS)])

    return k(col3, z32, ones32)


def _sc_aggr(t, row3, col3, z32):
    @functools.partial(
        pl.kernel,
        out_type=jax.ShapeDtypeStruct((NC, NP, H), jnp.float32),
        mesh=_mesh(),
        compiler_params=_SC_PARAMS,
        scratch_types=[
            pltpu.VMEM((CPW, CH), jnp.int32),
            pltpu.VMEM((CPW, CH), jnp.int32),
            pltpu.VMEM((CH, H), jnp.float32),
            pltpu.VMEM((CH, H), jnp.float32),
            pltpu.VMEM_SHARED((NP, H), jnp.float32),
            pltpu.VMEM_SHARED((NP, H), jnp.float32),
            pltpu.SemaphoreType.DMA,
            pltpu.SemaphoreType.DMA,
            pltpu.SemaphoreType.DMA,
            pltpu.SemaphoreType.DMA,
        ],
    )
    def k(t_hbm, row_hbm, col_hbm, z_hbm, out_hbm, rowv, colv, b0, b1,
          sh_t, sh_acc, gs0, gs1, ss0, ss1):
        cid = lax.axis_index("c")
        sid = lax.axis_index("s")
        wid = sid * NC + cid
        base = sid * RPS

        pltpu.sync_copy(t_hbm.at[pl.ds(base, RPS)], sh_t.at[pl.ds(base, RPS)])
        pltpu.sync_copy(z_hbm.at[pl.ds(base, RPS)], sh_acc.at[pl.ds(base, RPS)])
        pltpu.sync_copy(row_hbm.at[wid], rowv)
        pltpu.sync_copy(col_hbm.at[wid], colv)
        plsc.subcore_barrier()

        pltpu.async_copy(sh_t.at[rowv.at[0]], b0, gs0)
        pltpu.async_copy(sh_t.at[rowv.at[1]], b1, gs1)

        @pl.loop(0, CPW, step=2)
        def _(j):
            pltpu.make_async_copy(sh_t.at[rowv.at[j]], b0, gs0).wait()
            pltpu.async_copy(b0, sh_acc.at[colv.at[j]], ss0, add=True)
            pltpu.make_async_copy(sh_t.at[rowv.at[j + 1]], b1, gs1).wait()
            pltpu.async_copy(b1, sh_acc.at[colv.at[j + 1]], ss1, add=True)
            pltpu.make_async_copy(b0, sh_acc.at[colv.at[j]], ss0).wait()

            @pl.when(j + 2 < CPW)
            def _():
                pltpu.async_copy(sh_t.at[rowv.at[j + 2]], b0, gs0)

            pltpu.make_async_copy(b1, sh_acc.at[colv.at[j + 1]], ss1).wait()

            @pl.when(j + 3 < CPW)
            def _():
                pltpu.async_copy(sh_t.at[rowv.at[j + 3]], b1, gs1)

        plsc.subcore_barrier()
        pltpu.sync_copy(sh_acc.at[pl.ds(base, RPS)],
                        out_hbm.at[cid, pl.ds(base, RPS)])

    return k(t, row3, col3, z32)


def _whole(shape):
    nd = len(shape)
    return pl.BlockSpec(shape, lambda i, _nd=nd: (0,) * _nd)


def _rows(last):
    return pl.BlockSpec((BLK, last), lambda i: (i, 0))


def _pair(last):
    return pl.BlockSpec((NC, BLK, last), lambda i: (0, i, 0))


def _dis_of(c):
    return lax.rsqrt(1.0 + c[0] + c[1])


def _layernorm(y, gmean, lw, lb):
    mu = _dot_exact(y, gmean)
    d = y - mu
    var = _dot_exact(d * d, gmean)
    return d * lax.rsqrt(var + 1e-5) * lw + lb


def _tc0_body(x_ref, cnt_ref, inW4, inb, c_in, b_in, msgW4, c_msg, b_msg,
              h_ref, t_ref):
    h = _dot(x_ref[...], inW4[...]) + inb[...]
    h = _spline(h, c_in[...], b_in[...])
    dis = _dis_of(cnt_ref[...])
    h_ref[...] = h
    t_ref[...] = dis * _spline(_dot(h, msgW4[...]), c_msg[...], b_msg[...])


def _update(h, p, t, dis, selfW4, c_upd, b_upd, outW4, gmean, lw, lb):
    comb = dis * (p[0] + p[1] + t) + _dot(h, selfW4)
    u = _spline(comb, c_upd, b_upd)
    return h + _layernorm(_dot(u, outW4), gmean[...], lw, lb)


def _tcmid_body(h_ref, t_ref, p_ref, cnt_ref, gmean, selfW4, c_upd, b_upd,
                outW4, lw, lb, msgW4b, c_msg2, b_msg2, hn_ref, t2_ref):
    dis = _dis_of(cnt_ref[...])
    hn = _update(h_ref[...], p_ref[...], t_ref[...], dis, selfW4[...],
                 c_upd[...], b_upd[...], outW4[...], gmean, lw[...], lb[...])
    hn_ref[...] = hn
    t2_ref[...] = dis * _spline(_dot(hn, msgW4b[...]), c_msg2[...], b_msg2[...])


_LMASK = None


def _tcfin_body(h_ref, t_ref, p_ref, cnt_ref, bt_ref, bl_ref, gmean, selfW4,
                c_upd, b_upd, outW4, lw, lb, roW1t, rob1, ck1, bk1, roW2t,
                rob2, ck2, bk2, w3row, b3s, out_ref, ssum, scnt, smax):
    i = pl.program_id(0)

    @pl.when(i == 0)
    def _():
        ssum[...] = jnp.zeros_like(ssum)
        scnt[...] = jnp.zeros_like(scnt)
        smax[...] = jnp.full_like(smax, -jnp.inf)

    dis = _dis_of(cnt_ref[...])
    hn = _update(h_ref[...], p_ref[...], t_ref[...], dis, selfW4[...],
                 c_upd[...], b_upd[...], outW4[...], gmean, lw[...], lb[...])

    segs = lax.broadcasted_iota(jnp.int32, (16, BLK), 0)
    lanes = lax.broadcasted_iota(jnp.int32, (1, PK * H), 1) // H
    ones_rl = jnp.ones((BLK, PK * H), jnp.float32)
    acc_s = jnp.zeros((16, PK * H), jnp.float32)
    acc_c = jnp.zeros((16, PK * H), jnp.float32)
    for g in range(PK):
        brow_g = bt_ref[0, g:g + 1, :]
        Mg = (brow_g == segs).astype(jnp.float32)
        lmask_g = (lanes == g).astype(jnp.float32)
        acc_s = acc_s + _dot_exact(Mg, hn * lmask_g)
        acc_c = acc_c + _dot_exact(Mg, ones_rl * lmask_g)
    ssum[...] += acc_s
    scnt[...] += acc_c

    bl = bl_ref[...]
    for s in range(16):
        mg = jnp.max(jnp.where(bl == s, hn, -jnp.inf), axis=0, keepdims=True)
        smax[s:s + 1, :] = jnp.maximum(smax[s:s + 1, :], mg)

    @pl.when(i == GRID - 1)
    def _():
        ss = ssum[...]
        sc = scnt[...]
        sm = smax[...]
        sum32 = (ss[:, 0:H] + ss[:, H:2 * H] + ss[:, 2 * H:3 * H]
                 + ss[:, 3 * H:4 * H])
        cnt32 = (sc[:, 0:H] + sc[:, H:2 * H] + sc[:, 2 * H:3 * H]
                 + sc[:, 3 * H:4 * H])
        max32 = jnp.maximum(jnp.maximum(sm[:, 0:H], sm[:, H:2 * H]),
                            jnp.maximum(sm[:, 2 * H:3 * H], sm[:, 3 * H:4 * H]))
        mean = sum32 / jnp.maximum(cnt32, 1.0)
        mx = jnp.where(jnp.isfinite(max32), max32, 0.0)
        g64 = jnp.concatenate([mean, mx], axis=1)
        h1 = _spline(jnp.tile(_dot(g64, roW1t[...]) + rob1[...], (1, PK)),
                     ck1[...], bk1[...])[:, 0:H]
        h2 = _spline(jnp.tile(_dot(h1, roW2t[...]) + rob2[...], (1, 8)),
                     ck2[...], bk2[...])[:, 0:16]
        o = jnp.sum(h2 * w3row[...], axis=1, keepdims=True) + b3s[...]
        out_ref[...] = jnp.broadcast_to(jax.nn.sigmoid(o), (16, 16))


def _tc0(x4, cnt, inW4, inb, c_in, b_in, msgW4, c_msg, b_msg):
    return pl.pallas_call(
        _tc0_body,
        grid=(GRID,),
        in_specs=[_rows(PK * F), _pair(PK * H), _whole((PK * F, PK * H)),
                  _whole((1, PK * H)), _whole((11, PK * H)),
                  _whole((1, PK * H)), _whole((PK * H, PK * H)),
                  _whole((11, PK * H)), _whole((1, PK * H))],
        out_specs=[_rows(PK * H), _rows(PK * H)],
        out_shape=[jax.ShapeDtypeStruct((PR, PK * H), jnp.float32),
                   jax.ShapeDtypeStruct((PR, PK * H), jnp.float32)],
    )(x4, cnt, inW4, inb, c_in, b_in, msgW4, c_msg, b_msg)


def _tcmid(h, t, p, cnt, gmean, selfW4, c_upd, b_upd, outW4, lw, lb,
           msgW4b, c_msg2, b_msg2):
    return pl.pallas_call(
        _tcmid_body,
        grid=(GRID,),
        in_specs=[_rows(PK * H), _rows(PK * H), _pair(PK * H), _pair(PK * H),
                  _whole((PK * H, PK * H)), _whole((PK * H, PK * H)),
                  _whole((11, PK * H)), _whole((1, PK * H)),
                  _whole((PK * H, PK * H)), _whole((1, PK * H)),
                  _whole((1, PK * H)), _whole((PK * H, PK * H)),
                  _whole((11, PK * H)), _whole((1, PK * H))],
        out_specs=[_rows(PK * H), _rows(PK * H)],
        out_shape=[jax.ShapeDtypeStruct((PR, PK * H), jnp.float32),
                   jax.ShapeDtypeStruct((PR, PK * H), jnp.float32)],
    )(h, t, p, cnt, gmean, selfW4, c_upd, b_upd, outW4, lw, lb,
      msgW4b, c_msg2, b_msg2)


def _tcfin(h, t, p, cnt, bt, bl, gmean, selfW4, c_upd, b_upd, outW4, lw, lb,
           roW1t, rob1, ck1, bk1, roW2t, rob2, ck2, bk2, w3row, b3s):
    return pl.pallas_call(
        _tcfin_body,
        grid=(GRID,),
        in_specs=[_rows(PK * H), _rows(PK * H), _pair(PK * H), _pair(PK * H),
                  pl.BlockSpec((1, PK, BLK), lambda i: (i, 0, 0)),
                  _rows(PK * H),
                  _whole((PK * H, PK * H)), _whole((PK * H, PK * H)),
                  _whole((11, PK * H)), _whole((1, PK * H)),
                  _whole((PK * H, PK * H)), _whole((1, PK * H)),
                  _whole((1, PK * H)),
                  _whole((2 * H, H)), _whole((1, H)), _whole((11, PK * H)),
                  _whole((1, PK * H)),
                  _whole((H, 16)), _whole((1, 16)), _whole((11, PK * H)),
                  _whole((1, PK * H)),
                  _whole((1, 16)), _whole((1, 1))],
        out_specs=[pl.BlockSpec((16, 16), lambda i: (0, 0))],
        out_shape=[jax.ShapeDtypeStruct((16, 16), jnp.float32)],
        scratch_shapes=[pltpu.VMEM((16, PK * H), jnp.float32),
                        pltpu.VMEM((16, PK * H), jnp.float32),
                        pltpu.VMEM((16, PK * H), jnp.float32)],
    )(h, t, p, cnt, bt, bl, gmean, selfW4, c_upd, b_upd, outW4, lw, lb,
      roW1t, rob1, ck1, bk1, roW2t, rob2, ck2, bk2, w3row, b3s)


def _bd4(w):
    return jnp.kron(jnp.eye(PK, dtype=jnp.float32), w)


def kernel(x, edge_index, batch, params):
    p = params
    xp = jnp.zeros((NP, F), jnp.float32).at[:N].set(x)
    x4 = xp.reshape(PR, PK * F)
    row = edge_index[0].astype(jnp.int32)
    col = edge_index[1].astype(jnp.int32)
    row3 = jnp.full((EP,), N, jnp.int32).at[:E].set(row).reshape(NW, CPW, CH)
    col3 = jnp.full((EP,), N, jnp.int32).at[:E].set(col).reshape(NW, CPW, CH)
    bpad = jnp.full((NP,), 16, jnp.int32).at[:N].set(batch.astype(jnp.int32))
    bt = bpad.reshape(PR, PK).T.reshape(PK, GRID, BLK).transpose(1, 0, 2)
    bl = jnp.repeat(bpad.reshape(PR, PK), H, axis=1)
    z32 = jnp.zeros((NP, H), jnp.float32)
    ones32 = jnp.ones((CH, H), jnp.float32)
    gmean = _bd4(jnp.full((H, H), 1.0 / H, jnp.float32))

    c_in = jnp.tile(
        p['in_coeff'][:, None].astype(jnp.bfloat16).astype(jnp.float32),
        (1, PK * H))
    b_in = jnp.broadcast_to(jnp.abs(p['in_blend']),
                            (1, PK * H)).astype(jnp.float32)
    sp_msg = [_spline_tab(p['conv_msg_coeff'][l], p['conv_msg_blend'][l])
              for l in range(3)]
    sp_upd = [_spline_tab(p['conv_upd_coeff'][l], p['conv_upd_blend'][l])
              for l in range(3)]
    msgW4 = [_bd4(p['conv_msg_W'][l].T) for l in range(3)]
    selfW4 = [_bd4(p['conv_self_W'][l].T) for l in range(3)]
    outW4 = [_bd4(p['conv_out_W'][l].T) for l in range(3)]
    lw = [jnp.tile(p['conv_ln_w'][l][None, :], (1, PK)) for l in range(3)]
    lb = [jnp.tile(p['conv_ln_b'][l][None, :], (1, PK)) for l in range(3)]

    cnt = _sc_count(col3, z32, ones32)
    cnt4 = cnt.reshape(NC, PR, PK * H)

    h, t = _tc0(x4, cnt4, _bd4(p['in_W'].T),
                jnp.tile(p['in_b'][None, :], (1, PK)), c_in, b_in,
                msgW4[0], *sp_msg[0])

    for l in range(2):
        pt = _sc_aggr(t.reshape(NP, H), row3, col3, z32)
        h, t = _tcmid(h, t, pt.reshape(NC, PR, PK * H), cnt4, gmean,
                      selfW4[l], *sp_upd[l], outW4[l], lw[l], lb[l],
                      msgW4[l + 1], *sp_msg[l + 1])

    pt = _sc_aggr(t.reshape(NP, H), row3, col3, z32)
    ck1, bk1 = _spline_tab(p['ro_k1_coeff'], p['ro_k1_blend'])
    ck2raw = p['ro_k2_coeff'].T.astype(jnp.bfloat16).astype(jnp.float32)
    ck2 = jnp.tile(ck2raw, (1, 8))
    bk2 = jnp.tile(jnp.abs(p['ro_k2_blend'])[None, :].astype(jnp.float32),
                   (1, 8))
    out2d = _tcfin(h, t, pt.reshape(NC, PR, PK * H), cnt4, bt, bl, gmean,
                   selfW4[2], *sp_upd[2], outW4[2], lw[2], lb[2],
                   p['ro_W1'].T, p['ro_b1'][None, :], ck1, bk1,
                   p['ro_W2'].T, p['ro_b2'][None, :], ck2, bk2,
                   p['ro_W3'], p['ro_b3'][None, :])[0]
    return out2d[:, 0]
